# baseline (speedup 1.0000x reference)
import functools

import jax
import jax.numpy as jnp
from jax import lax
from jax.experimental import pallas as pl
from jax.experimental.pallas import tpu as pltpu, tpu_sc as plsc

N_NODES = 10000
N_PAD = 10240
N_EDGES = 160000
H = 32
FF = 256
HORIZON = 12

NC, NS = 2, 16
NW = NC * NS
EDGE_CHUNK = 8000


def _dot(w, x):
  return lax.dot_general(w, x, (((0,), (0,)), ((), ())),
                         preferred_element_type=jnp.float32)


def _pre_body(x3_ref, encw_ref, encb_ref, w0a_ref, w0b_ref, b0_ref,
              w1a_ref, w1b_ref, b1_ref, wsk_ref, bsk_ref,
              z2_ref, base_ref):
  x3 = x3_ref[...]
  encw = encw_ref[...]
  encb = encb_ref[...]
  h9 = encw * x3[0][None, :] + encb
  h10 = encw * x3[1][None, :] + encb
  h11 = encw * x3[2][None, :] + encb
  b0 = b0_ref[...]
  z1_10 = jnp.maximum(_dot(w0a_ref[...], h9) + _dot(w0b_ref[...], h10) + b0, 0.0)
  z1_11 = jnp.maximum(_dot(w0a_ref[...], h10) + _dot(w0b_ref[...], h11) + b0, 0.0)
  z2 = jnp.maximum(_dot(w1a_ref[...], z1_10) + _dot(w1b_ref[...], z1_11)
                   + b1_ref[...], 0.0)
  z2_ref[...] = z2
  base_ref[...] = h11 + _dot(wsk_ref[...], h11) + bsk_ref[...]


def _full_spec(a):
  return pl.BlockSpec(a.shape, lambda i: (0,) * a.ndim)


def _run_pre(x3, enc_W, enc_b, tc0_W, tc0_b, tc1_W, tc1_b, W_skip, b_skip):
  bn = 2048
  grid = (N_PAD // bn,)
  ws = [enc_W.reshape(H, 1), enc_b.reshape(H, 1),
        tc0_W[0], tc0_W[1], tc0_b.reshape(H, 1),
        tc1_W[0], tc1_W[1], tc1_b.reshape(H, 1),
        W_skip, b_skip.reshape(H, 1)]
  return pl.pallas_call(
      _pre_body,
      grid=grid,
      in_specs=[pl.BlockSpec((3, bn), lambda i: (0, i))]
      + [_full_spec(w) for w in ws],
      out_specs=[
          pl.BlockSpec((H, bn), lambda i: (0, i)),
          pl.BlockSpec((H, bn), lambda i: (0, i)),
      ],
      out_shape=[
          jax.ShapeDtypeStruct((H, N_PAD), jnp.float32),
          jax.ShapeDtypeStruct((H, N_PAD), jnp.float32),
      ],
  )(x3, *ws)


def _sc_body(z_hbm, sd_hbm, w_hbm, out_hbm,
             z_c, degf, degb, af1, af2, ab1, ab2,
             sd_c0, sd_c1, w_c0, w_c1, deg_part, deg_full, sems):
  sd_c = (sd_c0, sd_c1)
  w_c = (w_c0, w_c1)
  sid = lax.axis_index("s")
  wid = sid * NC + lax.axis_index("c")

  pltpu.sync_copy(z_hbm.at[wid], z_c)

  @plsc.parallel_loop(0, N_PAD // 16, unroll=4)
  def _(i):
    zv = jnp.zeros((16,), jnp.float32)
    sl = pl.ds(i * 16, 16)
    degf[sl] = zv
    degb[sl] = zv

  n_chunks = N_EDGES // EDGE_CHUNK
  n_inner = EDGE_CHUNK // 16
  NBUF = 2

  def issue(ci, b):
    base = ci * EDGE_CHUNK
    pltpu.async_copy(sd_hbm.at[pl.ds(base, EDGE_CHUNK)], sd_c[b],
                     sems.at[b, 0])
    pltpu.async_copy(w_hbm.at[pl.ds(base, EDGE_CHUNK)], w_c[b],
                     sems.at[b, 1])

  def drain(b):
    pltpu.make_async_copy(sd_hbm.at[pl.ds(0, EDGE_CHUNK)], sd_c[b],
                          sems.at[b, 0]).wait()
    pltpu.make_async_copy(w_hbm.at[pl.ds(0, EDGE_CHUNK)], w_c[b],
                          sems.at[b, 1]).wait()

  def prime():
    for b in range(NBUF):
      issue(b, b)

  def edge_pass(body):
    def group(g, _):
      for b in range(NBUF):
        ci = g * NBUF + b
        drain(b)

        @plsc.parallel_loop(0, n_inner, unroll=8)
        def _(j):
          sl = pl.ds(j * 16, 16)
          sd = sd_c[b][sl]
          body(lax.bitwise_and(sd, 65535), lax.shift_right_logical(sd, 16),
               w_c[b][sl])

        @pl.when(ci + NBUF < n_chunks)
        def _():
          issue(ci + NBUF, b)
      return 0
    lax.fori_loop(0, n_chunks // NBUF, group, 0)

  def scale_pair(a_ref, b_ref):
    @plsc.parallel_loop(0, N_PAD // 16, unroll=4)
    def _(i):
      sl = pl.ds(i * 16, 16)
      a_ref[sl] = a_ref[sl] * degf[sl]
      b_ref[sl] = b_ref[sl] * degb[sl]

  E_T = N_EDGES // NS
  base_e = sid * E_T
  for b, (off, ln) in enumerate(((0, EDGE_CHUNK), (EDGE_CHUNK, E_T - EDGE_CHUNK))):
    pltpu.sync_copy(sd_hbm.at[pl.ds(base_e + off, ln)],
                    sd_c[b].at[pl.ds(0, ln)])
    pltpu.sync_copy(w_hbm.at[pl.ds(base_e + off, ln)],
                    w_c[b].at[pl.ds(0, ln)])

    @plsc.parallel_loop(0, ln // 16, unroll=4)
    def _(j):
      sl = pl.ds(j * 16, 16)
      sd = sd_c[b][sl]
      w = w_c[b][sl]
      plsc.addupdate_scatter(degf, [lax.shift_right_logical(sd, 16)], w)
      plsc.addupdate_scatter(degb, [lax.bitwise_and(sd, 65535)], w)

  pltpu.sync_copy(degf, deg_part.at[sid, 0])
  pltpu.sync_copy(degb, deg_part.at[sid, 1])
  plsc.subcore_barrier()

  STR = N_PAD // NS
  for k in range(NS):
    pltpu.sync_copy(deg_part.at[k, 0, pl.ds(sid * STR, STR)],
                    af1.at[pl.ds(k * STR, STR)])
    pltpu.sync_copy(deg_part.at[k, 1, pl.ds(sid * STR, STR)],
                    ab1.at[pl.ds(k * STR, STR)])

  @plsc.parallel_loop(0, STR // 16, unroll=2)
  def _(j):
    accf = af1[pl.ds(j * 16, 16)]
    accb = ab1[pl.ds(j * 16, 16)]
    for k in range(1, NS):
      accf = accf + af1[pl.ds(k * STR + j * 16, 16)]
      accb = accb + ab1[pl.ds(k * STR + j * 16, 16)]
    accf = jnp.where(accf > 0.0, 1.0 / accf, 0.0)
    accb = jnp.where(accb > 0.0, 1.0 / accb, 0.0)
    osl = pl.ds(sid * STR + j * 16, 16)
    degf[osl] = accf
    degb[osl] = accb

  pltpu.sync_copy(degf.at[pl.ds(sid * STR, STR)],
                  deg_full.at[0, pl.ds(sid * STR, STR)])
  pltpu.sync_copy(degb.at[pl.ds(sid * STR, STR)],
                  deg_full.at[1, pl.ds(sid * STR, STR)])
  plsc.subcore_barrier()
  pltpu.sync_copy(deg_full.at[0], degf)
  pltpu.sync_copy(deg_full.at[1], degb)

  @plsc.parallel_loop(0, N_PAD // 16, unroll=4)
  def _(i):
    zv = jnp.zeros((16,), jnp.float32)
    sl = pl.ds(i * 16, 16)
    af1[sl] = zv
    af2[sl] = zv
    ab1[sl] = zv
    ab2[sl] = zv

  def hop1_body(s, d, w):
    plsc.addupdate_scatter(af1, [d], w * plsc.load_gather(z_c, [s]))
    plsc.addupdate_scatter(ab1, [s], w * plsc.load_gather(z_c, [d]))
  prime()
  edge_pass(hop1_body)
  prime()

  scale_pair(af1, ab1)

  cp1 = pltpu.async_copy(af1, out_hbm.at[0, wid], sems.at[0, 2])
  cp2 = pltpu.async_copy(ab1, out_hbm.at[2, wid], sems.at[1, 2])

  def hop2_body(s, d, w):
    plsc.addupdate_scatter(af2, [d], w * plsc.load_gather(af1, [s]))
    plsc.addupdate_scatter(ab2, [s], w * plsc.load_gather(ab1, [d]))
  edge_pass(hop2_body)
  scale_pair(af2, ab2)

  cp1.wait()
  cp2.wait()
  pltpu.sync_copy(af2, out_hbm.at[1, wid])
  pltpu.sync_copy(ab2, out_hbm.at[3, wid])


def _run_sc(z2, sd, w):
  f = pl.kernel(
      _sc_body,
      out_type=jax.ShapeDtypeStruct((4, H, N_PAD), jnp.float32),
      mesh=plsc.VectorSubcoreMesh(core_axis_name="c", subcore_axis_name="s"),
      scratch_types=[
          pltpu.VMEM((N_PAD,), jnp.float32),
          pltpu.VMEM((N_PAD,), jnp.float32),
          pltpu.VMEM((N_PAD,), jnp.float32),
          pltpu.VMEM((N_PAD,), jnp.float32),
          pltpu.VMEM((N_PAD,), jnp.float32),
          pltpu.VMEM((N_PAD,), jnp.float32),
          pltpu.VMEM((N_PAD,), jnp.float32),
          pltpu.VMEM((EDGE_CHUNK,), jnp.int32),
          pltpu.VMEM((EDGE_CHUNK,), jnp.int32),
          pltpu.VMEM((EDGE_CHUNK,), jnp.float32),
          pltpu.VMEM((EDGE_CHUNK,), jnp.float32),
          pltpu.VMEM_SHARED((NS, 2, N_PAD), jnp.float32),
          pltpu.VMEM_SHARED((2, N_PAD), jnp.float32),
          pltpu.SemaphoreType.DMA((2, 3)),
      ],
      compiler_params=pltpu.CompilerParams(needs_layout_passes=False),
  )
  return f(z2, sd, w)


def _post_body(z2_ref, dout_ref, base_ref, wd_ref, bd_ref,
               w1_ref, b1_ref, w2_ref, b2_ref, y_ref):
  wd = wd_ref[...]
  d = _dot(wd[0:H], z2_ref[...]) + bd_ref[...]
  dout = dout_ref[...]
  d += _dot(wd[H:2 * H], dout[0])
  d += _dot(wd[2 * H:3 * H], dout[1])
  d += _dot(wd[3 * H:4 * H], dout[2])
  d += _dot(wd[4 * H:5 * H], dout[3])
  hl = base_ref[...] + jnp.maximum(d, 0.0)
  y1 = jnp.maximum(_dot(w1_ref[...], hl) + b1_ref[...], 0.0)
  y_ref[...] = _dot(w2_ref[...], y1) + b2_ref[...]


def _run_post(z2, dout, base, W_diff, b_diff, W1, b1, W2, b2):
  bn = 2048
  grid = (N_PAD // bn,)
  ws = [W_diff, b_diff.reshape(H, 1), W1, b1.reshape(FF, 1),
        W2, b2.reshape(HORIZON, 1)]
  return pl.pallas_call(
      _post_body,
      grid=grid,
      in_specs=[
          pl.BlockSpec((H, bn), lambda i: (0, i)),
          pl.BlockSpec((4, H, bn), lambda i: (0, 0, i)),
          pl.BlockSpec((H, bn), lambda i: (0, i)),
      ] + [_full_spec(w) for w in ws],
      out_specs=pl.BlockSpec((HORIZON, bn), lambda i: (0, i)),
      out_shape=jax.ShapeDtypeStruct((HORIZON, N_PAD), jnp.float32),
  )(z2, dout, base, *ws)


def kernel(x, edge_index, edge_weight, enc_W, enc_b, W_skip, b_skip, tc0_W,
           tc0_b, tc1_W, tc1_b, W_diff, b_diff, W1, b1, W2, b2):
  x3 = x[0, -3:, :, 0]
  x3 = jnp.pad(x3, ((0, 0), (0, N_PAD - N_NODES)))
  z2, base = _run_pre(x3, enc_W, enc_b, tc0_W, tc0_b, tc1_W, tc1_b,
                      W_skip, b_skip)
  ei = edge_index.astype(jnp.int32)
  sd = ei[0] | (ei[1] << 16)
  dout = _run_sc(z2, sd, edge_weight)
  y = _run_post(z2, dout, base, W_diff, b_diff, W1, b1, W2, b2)
  return y[:, :N_NODES][None, :, :, None]


# speedup vs baseline: 1.0228x; 1.0228x over previous
import jax
import jax.numpy as jnp
from jax import lax
from jax.experimental import pallas as pl
from jax.experimental.pallas import tpu as pltpu, tpu_sc as plsc

N_NODES = 10000
N_PAD = 10240
N_EDGES = 160000
H = 32
FF = 256
HORIZON = 12

NC, NS = 2, 16
NW = NC * NS
EDGE_CHUNK = 8000


def _dot(w, x):
  return lax.dot_general(w, x, (((0,), (0,)), ((), ())),
                         preferred_element_type=jnp.float32)


def _pre_body(x3_ref, encw_ref, encb_ref, w0a_ref, w0b_ref, b0_ref,
              w1a_ref, w1b_ref, b1_ref, wsk_ref, bsk_ref,
              z2_ref, base_ref):
  x3 = x3_ref[...]
  encw = encw_ref[...]
  encb = encb_ref[...]
  h9 = encw * x3[0][None, :] + encb
  h10 = encw * x3[1][None, :] + encb
  h11 = encw * x3[2][None, :] + encb
  b0 = b0_ref[...]
  z1_10 = jnp.maximum(_dot(w0a_ref[...], h9) + _dot(w0b_ref[...], h10) + b0, 0.0)
  z1_11 = jnp.maximum(_dot(w0a_ref[...], h10) + _dot(w0b_ref[...], h11) + b0, 0.0)
  z2 = jnp.maximum(_dot(w1a_ref[...], z1_10) + _dot(w1b_ref[...], z1_11)
                   + b1_ref[...], 0.0)
  z2_ref[...] = z2
  base_ref[...] = h11 + _dot(wsk_ref[...], h11) + bsk_ref[...]


def _full_spec(a):
  return pl.BlockSpec(a.shape, lambda i: (0,) * a.ndim)


def _run_pre(x3, enc_W, enc_b, tc0_W, tc0_b, tc1_W, tc1_b, W_skip, b_skip):
  bn = 2048
  grid = (N_PAD // bn,)
  ws = [enc_W.reshape(H, 1), enc_b.reshape(H, 1),
        tc0_W[0], tc0_W[1], tc0_b.reshape(H, 1),
        tc1_W[0], tc1_W[1], tc1_b.reshape(H, 1),
        W_skip, b_skip.reshape(H, 1)]
  return pl.pallas_call(
      _pre_body,
      grid=grid,
      in_specs=[pl.BlockSpec((3, bn), lambda i: (0, i))]
      + [_full_spec(w) for w in ws],
      out_specs=[
          pl.BlockSpec((H, bn), lambda i: (0, i)),
          pl.BlockSpec((H, bn), lambda i: (0, i)),
      ],
      out_shape=[
          jax.ShapeDtypeStruct((H, N_PAD), jnp.float32),
          jax.ShapeDtypeStruct((H, N_PAD), jnp.float32),
      ],
  )(x3, *ws)


def _sc_body(z_hbm, sd_hbm, w_hbm, out_hbm,
             z_c, degf, degb, af1, af2, ab1, ab2,
             sd_c0, sd_c1, w_c0, w_c1, deg_part, deg_full, sems):
  sd_c = (sd_c0, sd_c1)
  w_c = (w_c0, w_c1)
  sid = lax.axis_index("s")
  wid = sid * NC + lax.axis_index("c")

  pltpu.sync_copy(z_hbm.at[wid], z_c)

  @plsc.parallel_loop(0, N_PAD // 16, unroll=4)
  def _(i):
    zv = jnp.zeros((16,), jnp.float32)
    sl = pl.ds(i * 16, 16)
    degf[sl] = zv
    degb[sl] = zv

  n_chunks = N_EDGES // EDGE_CHUNK
  n_inner = EDGE_CHUNK // 16
  NBUF = 2

  def issue(ci, b):
    base = ci * EDGE_CHUNK
    pltpu.async_copy(sd_hbm.at[pl.ds(base, EDGE_CHUNK)], sd_c[b],
                     sems.at[b, 0])
    pltpu.async_copy(w_hbm.at[pl.ds(base, EDGE_CHUNK)], w_c[b],
                     sems.at[b, 1])

  def drain(b):
    pltpu.make_async_copy(sd_hbm.at[pl.ds(0, EDGE_CHUNK)], sd_c[b],
                          sems.at[b, 0]).wait()
    pltpu.make_async_copy(w_hbm.at[pl.ds(0, EDGE_CHUNK)], w_c[b],
                          sems.at[b, 1]).wait()

  def prime():
    for b in range(NBUF):
      issue(b, b)

  def edge_pass(body):
    def group(g, _):
      for b in range(NBUF):
        ci = g * NBUF + b
        drain(b)

        @plsc.parallel_loop(0, n_inner, unroll=4)
        def _(j):
          sl = pl.ds(j * 16, 16)
          sd = sd_c[b][sl]
          body(lax.bitwise_and(sd, 65535), lax.shift_right_logical(sd, 16),
               w_c[b][sl])

        @pl.when(ci + NBUF < n_chunks)
        def _():
          issue(ci + NBUF, b)
      return 0
    lax.fori_loop(0, n_chunks // NBUF, group, 0)

  def scale_pair(a_ref, b_ref):
    @plsc.parallel_loop(0, N_PAD // 16, unroll=4)
    def _(i):
      sl = pl.ds(i * 16, 16)
      a_ref[sl] = a_ref[sl] * degf[sl]
      b_ref[sl] = b_ref[sl] * degb[sl]

  E_T = N_EDGES // NS
  base_e = sid * E_T
  for b, (off, ln) in enumerate(((0, EDGE_CHUNK), (EDGE_CHUNK, E_T - EDGE_CHUNK))):
    pltpu.sync_copy(sd_hbm.at[pl.ds(base_e + off, ln)],
                    sd_c[b].at[pl.ds(0, ln)])
    pltpu.sync_copy(w_hbm.at[pl.ds(base_e + off, ln)],
                    w_c[b].at[pl.ds(0, ln)])

    @plsc.parallel_loop(0, ln // 16, unroll=4)
    def _(j):
      sl = pl.ds(j * 16, 16)
      sd = sd_c[b][sl]
      w = w_c[b][sl]
      plsc.addupdate_scatter(degf, [lax.shift_right_logical(sd, 16)], w)
      plsc.addupdate_scatter(degb, [lax.bitwise_and(sd, 65535)], w)

  pltpu.sync_copy(degf, deg_part.at[sid, 0])
  pltpu.sync_copy(degb, deg_part.at[sid, 1])
  plsc.subcore_barrier()

  STR = N_PAD // NS
  for k in range(NS):
    pltpu.sync_copy(deg_part.at[k, 0, pl.ds(sid * STR, STR)],
                    af1.at[pl.ds(k * STR, STR)])
    pltpu.sync_copy(deg_part.at[k, 1, pl.ds(sid * STR, STR)],
                    ab1.at[pl.ds(k * STR, STR)])

  @plsc.parallel_loop(0, STR // 16, unroll=2)
  def _(j):
    accf = af1[pl.ds(j * 16, 16)]
    accb = ab1[pl.ds(j * 16, 16)]
    for k in range(1, NS):
      accf = accf + af1[pl.ds(k * STR + j * 16, 16)]
      accb = accb + ab1[pl.ds(k * STR + j * 16, 16)]
    accf = jnp.where(accf > 0.0, 1.0 / accf, 0.0)
    accb = jnp.where(accb > 0.0, 1.0 / accb, 0.0)
    osl = pl.ds(sid * STR + j * 16, 16)
    degf[osl] = accf
    degb[osl] = accb

  pltpu.sync_copy(degf.at[pl.ds(sid * STR, STR)],
                  deg_full.at[0, pl.ds(sid * STR, STR)])
  pltpu.sync_copy(degb.at[pl.ds(sid * STR, STR)],
                  deg_full.at[1, pl.ds(sid * STR, STR)])
  plsc.subcore_barrier()
  pltpu.sync_copy(deg_full.at[0], degf)
  pltpu.sync_copy(deg_full.at[1], degb)

  @plsc.parallel_loop(0, N_PAD // 16, unroll=4)
  def _(i):
    zv = jnp.zeros((16,), jnp.float32)
    sl = pl.ds(i * 16, 16)
    af1[sl] = zv
    af2[sl] = zv
    ab1[sl] = zv
    ab2[sl] = zv

  def hop1_body(s, d, w):
    plsc.addupdate_scatter(af1, [d], w * plsc.load_gather(z_c, [s]))
    plsc.addupdate_scatter(ab1, [s], w * plsc.load_gather(z_c, [d]))
  prime()
  edge_pass(hop1_body)
  prime()

  scale_pair(af1, ab1)

  cp1 = pltpu.async_copy(af1, out_hbm.at[0, wid], sems.at[0, 2])
  cp2 = pltpu.async_copy(ab1, out_hbm.at[2, wid], sems.at[1, 2])

  def hop2_body(s, d, w):
    plsc.addupdate_scatter(af2, [d], w * plsc.load_gather(af1, [s]))
    plsc.addupdate_scatter(ab2, [s], w * plsc.load_gather(ab1, [d]))
  edge_pass(hop2_body)
  scale_pair(af2, ab2)

  cp1.wait()
  cp2.wait()
  pltpu.sync_copy(af2, out_hbm.at[1, wid])
  pltpu.sync_copy(ab2, out_hbm.at[3, wid])


def _run_sc(z2, sd, w):
  f = pl.kernel(
      _sc_body,
      out_type=jax.ShapeDtypeStruct((4, H, N_PAD), jnp.float32),
      mesh=plsc.VectorSubcoreMesh(core_axis_name="c", subcore_axis_name="s"),
      scratch_types=[
          pltpu.VMEM((N_PAD,), jnp.float32),
          pltpu.VMEM((N_PAD,), jnp.float32),
          pltpu.VMEM((N_PAD,), jnp.float32),
          pltpu.VMEM((N_PAD,), jnp.float32),
          pltpu.VMEM((N_PAD,), jnp.float32),
          pltpu.VMEM((N_PAD,), jnp.float32),
          pltpu.VMEM((N_PAD,), jnp.float32),
          pltpu.VMEM((EDGE_CHUNK,), jnp.int32),
          pltpu.VMEM((EDGE_CHUNK,), jnp.int32),
          pltpu.VMEM((EDGE_CHUNK,), jnp.float32),
          pltpu.VMEM((EDGE_CHUNK,), jnp.float32),
          pltpu.VMEM_SHARED((NS, 2, N_PAD), jnp.float32),
          pltpu.VMEM_SHARED((2, N_PAD), jnp.float32),
          pltpu.SemaphoreType.DMA((2, 3)),
      ],
      compiler_params=pltpu.CompilerParams(needs_layout_passes=False),
  )
  return f(z2, sd, w)


def _post_body(z2_ref, dout_ref, base_ref, wd_ref, bd_ref,
               w1_ref, b1_ref, w2_ref, b2_ref, y_ref):
  wd = wd_ref[...]
  d = _dot(wd[0:H], z2_ref[...]) + bd_ref[...]
  dout = dout_ref[...]
  d += _dot(wd[H:2 * H], dout[0])
  d += _dot(wd[2 * H:3 * H], dout[1])
  d += _dot(wd[3 * H:4 * H], dout[2])
  d += _dot(wd[4 * H:5 * H], dout[3])
  hl = base_ref[...] + jnp.maximum(d, 0.0)
  y1 = jnp.maximum(_dot(w1_ref[...], hl) + b1_ref[...], 0.0)
  y_ref[...] = _dot(w2_ref[...], y1) + b2_ref[...]


def _run_post(z2, dout, base, W_diff, b_diff, W1, b1, W2, b2):
  bn = 2048
  grid = (N_PAD // bn,)
  ws = [W_diff, b_diff.reshape(H, 1), W1, b1.reshape(FF, 1),
        W2, b2.reshape(HORIZON, 1)]
  return pl.pallas_call(
      _post_body,
      grid=grid,
      in_specs=[
          pl.BlockSpec((H, bn), lambda i: (0, i)),
          pl.BlockSpec((4, H, bn), lambda i: (0, 0, i)),
          pl.BlockSpec((H, bn), lambda i: (0, i)),
      ] + [_full_spec(w) for w in ws],
      out_specs=pl.BlockSpec((HORIZON, bn), lambda i: (0, i)),
      out_shape=jax.ShapeDtypeStruct((HORIZON, N_PAD), jnp.float32),
  )(z2, dout, base, *ws)


def kernel(x, edge_index, edge_weight, enc_W, enc_b, W_skip, b_skip, tc0_W,
           tc0_b, tc1_W, tc1_b, W_diff, b_diff, W1, b1, W2, b2):
  x3 = x[0, -3:, :, 0]
  x3 = jnp.pad(x3, ((0, 0), (0, N_PAD - N_NODES)))
  z2, base = _run_pre(x3, enc_W, enc_b, tc0_W, tc0_b, tc1_W, tc1_b,
                      W_skip, b_skip)
  ei = edge_index.astype(jnp.int32)
  sd = ei[0] | (ei[1] << 16)
  dout = _run_sc(z2, sd, edge_weight)
  y = _run_post(z2, dout, base, W_diff, b_diff, W1, b1, W2, b2)
  return y[:, :N_NODES][None, :, :, None]
